# baseline (speedup 1.0000x reference)
import functools

import jax
import jax.numpy as jnp
from jax import lax
from jax.experimental import pallas as pl
from jax.experimental.pallas import tpu as pltpu
from jax.experimental.pallas import tpu_sc as plsc

_NC = 2
_NS = 16
_NW = _NC * _NS
_L = 16
_BLK = 128


@functools.lru_cache(maxsize=None)
def _build(B, H, D, DP, NBUF):
    n_bb = B // _BLK
    bb_per_w = n_bb // _NW
    n_chunks = H * bb_per_w
    DG = D // 8
    mesh = plsc.VectorSubcoreMesh(core_axis_name="c", subcore_axis_name="s")

    @functools.partial(
        pl.kernel,
        mesh=mesh,
        out_type=jax.ShapeDtypeStruct((H, DG, n_bb, 8, _BLK), jnp.float32),
        compiler_params=pltpu.CompilerParams(use_tc_tiling_on_sc=False,
                                             needs_layout_passes=False),
        scratch_types=[
            pltpu.VMEM((bb_per_w, H, _BLK), jnp.int32),
            pltpu.VMEM((NBUF, _BLK, DP), jnp.float32),
            pltpu.VMEM((2, D, _BLK), jnp.float32),
            pltpu.SemaphoreType.DMA((NBUF,)),
            pltpu.SemaphoreType.DMA((2,)),
        ],
    )
    def emb(idx_hbm, table_hbm, out_hbm, idx_v, rows_v, tr_v, gsem, wsem):
        wid = lax.axis_index("s") * _NC + lax.axis_index("c")
        bb0 = wid * bb_per_w
        pltpu.sync_copy(idx_hbm.at[pl.ds(bb0, bb_per_w)], idx_v)

        def gather(h, k):
            pltpu.async_copy(table_hbm.at[idx_v.at[k, h]], rows_v.at[k],
                             gsem.at[k])

        def gather_wait(k):
            pltpu.make_async_copy(table_hbm.at[idx_v.at[0, 0]],
                                  rows_v.at[k], gsem.at[k]).wait()

        def wb(h, k):
            tbuf = k % 2
            for dg in range(DG):
                pltpu.async_copy(tr_v.at[tbuf, pl.ds(dg * 8, 8)],
                                 out_hbm.at[h, dg, bb0 + k], wsem.at[tbuf])

        def wb_wait(k):
            tbuf = k % 2
            for dg in range(DG):
                pltpu.make_async_copy(tr_v.at[tbuf, pl.ds(dg * 8, 8)],
                                      out_hbm.at[0, 0, 0], wsem.at[tbuf]).wait()

        lanes = lax.iota(jnp.int32, _L)

        def transpose(k):
            rows = rows_v.at[k]
            tbuf = k % 2

            def per_d(d, carry):
                dsplat = jnp.full((_L,), 0, jnp.int32) + d
                for g in range(_BLK // _L):
                    bl = lanes + (g * _L)
                    val = plsc.load_gather(rows, [bl, dsplat])
                    tr_v[tbuf, d, pl.ds(g * _L, _L)] = val
                return carry

            lax.fori_loop(0, D, per_d, 0)

        for k in range(bb_per_w):
            gather(0, k)

        def step(h, carry):
            for k in range(bb_per_w):
                if k < 2:
                    @pl.when(h > 0)
                    def _():
                        wb_wait(k)
                else:
                    wb_wait(k)
                gather_wait(k)
                transpose(k)
                wb(h, k)

                @pl.when(h < H - 1)
                def _():
                    gather(h + 1, k)
            return carry

        lax.fori_loop(0, H, step, 0)

        wb_wait(2)
        wb_wait(3)

    return emb


def kernel(X, table):
    B, H = X.shape
    V, D = table.shape
    DP = 128
    idx = X.T.reshape(H, B // _BLK, _BLK).transpose(1, 0, 2)
    table_p = jnp.pad(table, ((0, 0), (0, DP - D)))
    out5 = _build(B, H, D, DP, 4)(idx, table_p)
    return out5.transpose(2, 4, 0, 1, 3).reshape(B, H, D)


# speedup vs baseline: 1.4333x; 1.4333x over previous
import functools

import jax
import jax.numpy as jnp
from jax import lax
from jax.experimental import pallas as pl
from jax.experimental.pallas import tpu as pltpu
from jax.experimental.pallas import tpu_sc as plsc

_NC = 2
_NS = 16
_NW = _NC * _NS
_L = 16
_BLK = 128


@functools.lru_cache(maxsize=None)
def _build(B, H, D, DP, NBUF):
    n_bb = B // _BLK
    bb_per_w = n_bb // _NW
    n_chunks = H * bb_per_w
    DG = D // 8
    mesh = plsc.VectorSubcoreMesh(core_axis_name="c", subcore_axis_name="s")

    @functools.partial(
        pl.kernel,
        mesh=mesh,
        out_type=jax.ShapeDtypeStruct((H, DG, n_bb, 8, _BLK), jnp.float32),
        compiler_params=pltpu.CompilerParams(use_tc_tiling_on_sc=False,
                                             needs_layout_passes=False),
        scratch_types=[
            pltpu.VMEM((bb_per_w, H, _BLK), jnp.int32),
            pltpu.VMEM((NBUF, _BLK, DP), jnp.float32),
            pltpu.VMEM((2, D, _BLK), jnp.float32),
            pltpu.SemaphoreType.DMA((NBUF,)),
            pltpu.SemaphoreType.DMA((2,)),
        ],
    )
    def emb(idx_hbm, table_hbm, out_hbm, idx_v, rows_v, tr_v, gsem, wsem):
        wid = lax.axis_index("s") * _NC + lax.axis_index("c")
        bb0 = wid * bb_per_w
        pltpu.sync_copy(idx_hbm.at[pl.ds(bb0, bb_per_w)], idx_v)

        def gather(h, k):
            pltpu.async_copy(table_hbm.at[idx_v.at[k, h]], rows_v.at[k],
                             gsem.at[k])

        def gather_wait(k):
            pltpu.make_async_copy(table_hbm.at[idx_v.at[0, 0]],
                                  rows_v.at[k], gsem.at[k]).wait()

        def wb(h, k):
            tbuf = k % 2
            for dg in range(DG):
                pltpu.async_copy(tr_v.at[tbuf, pl.ds(dg * 8, 8)],
                                 out_hbm.at[h, dg, bb0 + k], wsem.at[tbuf])

        def wb_wait(k):
            tbuf = k % 2
            for dg in range(DG):
                pltpu.make_async_copy(tr_v.at[tbuf, pl.ds(dg * 8, 8)],
                                      out_hbm.at[0, 0, 0], wsem.at[tbuf]).wait()

        lanes = lax.iota(jnp.int32, _L)

        def transpose(k):
            rows = rows_v.at[k]
            tbuf = k % 2

            @plsc.parallel_loop(0, D, unroll=8)
            def per_d(d):
                dsplat = jnp.full((_L,), 0, jnp.int32) + d
                for g in range(_BLK // _L):
                    bl = lanes + (g * _L)
                    val = plsc.load_gather(rows, [bl, dsplat])
                    tr_v[tbuf, d, pl.ds(g * _L, _L)] = val

        for k in range(bb_per_w):
            gather(0, k)

        def step(h, carry):
            for k in range(bb_per_w):
                if k < 2:
                    @pl.when(h > 0)
                    def _():
                        wb_wait(k)
                else:
                    wb_wait(k)
                gather_wait(k)
                transpose(k)
                wb(h, k)

                @pl.when(h < H - 1)
                def _():
                    gather(h + 1, k)
            return carry

        lax.fori_loop(0, H, step, 0)

        wb_wait(2)
        wb_wait(3)

    return emb


def kernel(X, table):
    B, H = X.shape
    V, D = table.shape
    DP = 128
    idx = X.T.reshape(H, B // _BLK, _BLK).transpose(1, 0, 2)
    table_p = jnp.pad(table, ((0, 0), (0, DP - D)))
    out5 = _build(B, H, D, DP, 4)(idx, table_p)
    return out5.transpose(2, 4, 0, 1, 3).reshape(B, H, D)
